# baseline (speedup 1.0000x reference)
import functools

import jax
import jax.numpy as jnp
from jax import lax
from jax.experimental import pallas as pl
from jax.experimental.pallas import tpu as pltpu
from jax.experimental.pallas import tpu_sc as plsc

N = 10000
E = 320000
D = 128
L = 3

NC = 2
NS = 16
NW = NC * NS
NP = 10240
RPW = NP // NS
ZBR = 32

EPW = E // NW
EPWP = 10240
PADE = EPWP - EPW
CK = 64
CH = EPWP // CK
NSLOT = 4


def _sc_body(h_hbm, src_hbm, dst_hbm, agg_out, cnt_out, src_r, dst_r,
             rows_r, zb_v, ones_v, zbc_v, agg_sh, cnt_sh,
             sem_i, sem_g, sem_s):
    cid = lax.axis_index("c")
    sid = lax.axis_index("s")
    wid = cid * NS + sid

    def _zrow(r, _):
        for c in range(D // 16):
            zb_v[r, pl.ds(c * 16, 16)] = jnp.zeros((16,), jnp.float32)
        return 0
    lax.fori_loop(0, ZBR, _zrow, 0)

    def _zcnt(k, _):
        zbc_v[pl.ds(k * 16, 16)] = jnp.zeros((16,), jnp.float32)
        return 0
    lax.fori_loop(0, RPW // 16, _zcnt, 0)

    def _ofill(k, _):
        ones_v[pl.ds(k * 16, 16)] = jnp.ones((16,), jnp.float32)
        return 0
    lax.fori_loop(0, CK // 16, _ofill, 0)

    for k in range(RPW // ZBR):
        pltpu.sync_copy(zb_v, agg_sh.at[pl.ds(sid * RPW + k * ZBR, ZBR)])
    pltpu.sync_copy(zbc_v, cnt_sh.at[pl.ds(sid * RPW, RPW)])
    plsc.subcore_barrier()

    def _idx_load(jj, s):
        off = wid * EPWP + jj * CK
        pltpu.async_copy(src_hbm.at[pl.ds(off, CK)], src_r.at[s],
                         sem_i.at[s])
        pltpu.async_copy(dst_hbm.at[pl.ds(off, CK)], dst_r.at[s],
                         sem_i.at[s])

    def _idx_wait(s):
        pltpu.make_async_copy(src_hbm.at[pl.ds(0, CK)], src_r.at[s],
                              sem_i.at[s]).wait()
        pltpu.make_async_copy(dst_hbm.at[pl.ds(0, CK)], dst_r.at[s],
                              sem_i.at[s]).wait()

    def _gather(s):
        pltpu.async_copy(h_hbm.at[src_r.at[s]], rows_r.at[s], sem_g.at[s])

    def _gather_wait(s):
        pltpu.make_async_copy(h_hbm.at[src_r.at[s]], rows_r.at[s],
                              sem_g.at[s]).wait()

    def _scatter(s):
        pltpu.async_copy(rows_r.at[s], agg_sh.at[dst_r.at[s]], sem_s.at[s],
                         add=True)
        pltpu.async_copy(ones_v, cnt_sh.at[dst_r.at[s]], sem_s.at[s],
                         add=True)

    def _scatter_wait(s):
        pltpu.make_async_copy(rows_r.at[s], agg_sh.at[dst_r.at[s]],
                              sem_s.at[s]).wait()
        pltpu.make_async_copy(ones_v, cnt_sh.at[dst_r.at[s]],
                              sem_s.at[s]).wait()

    for t in range(3):
        pltpu.sync_copy(src_hbm.at[pl.ds(wid * EPWP + t * CK, CK)],
                        src_r.at[t])
        pltpu.sync_copy(dst_hbm.at[pl.ds(wid * EPWP + t * CK, CK)],
                        dst_r.at[t])
    _gather(0)
    _gather(1)

    def _group(g, _):
        for b in range(NSLOT):
            j = g * NSLOT + b
            _gather_wait(b)
            _scatter(b)
            bg = (b + 2) % NSLOT
            @pl.when(j + 2 < CH)
            def _c():
                @pl.when(j >= 1)
                def _cw():
                    _idx_wait(bg)
                _gather(bg)
            bi = (b + 3) % NSLOT
            @pl.when(j + 3 < CH)
            def _d():
                @pl.when(j >= 1)
                def _dw():
                    _scatter_wait(bi)
                _idx_load(j + 3, bi)
        return 0
    lax.fori_loop(0, CH // NSLOT, _group, 0)
    for b in range(NSLOT):
        _scatter_wait(b)

    plsc.subcore_barrier()
    pltpu.sync_copy(agg_sh.at[pl.ds(sid * RPW, RPW)],
                    agg_out.at[cid, pl.ds(sid * RPW, RPW)])
    pltpu.sync_copy(cnt_sh.at[pl.ds(sid * RPW, RPW)], zbc_v)
    pltpu.sync_copy(zbc_v, cnt_out.at[pl.ds(cid * NP + sid * RPW, RPW)])


def _make_sc_agg():
    mesh = plsc.VectorSubcoreMesh(core_axis_name="c", subcore_axis_name="s")
    out_type = (jax.ShapeDtypeStruct((NC, NP, D), jnp.float32),
                jax.ShapeDtypeStruct((NC * NP,), jnp.float32))
    scratch = [
        pltpu.VMEM((NSLOT, CK), jnp.int32),
        pltpu.VMEM((NSLOT, CK), jnp.int32),
        pltpu.VMEM((NSLOT, CK, D), jnp.float32),
        pltpu.VMEM((ZBR, D), jnp.float32),
        pltpu.VMEM((CK,), jnp.float32),
        pltpu.VMEM((RPW,), jnp.float32),
        pltpu.VMEM_SHARED((NP, D), jnp.float32),
        pltpu.VMEM_SHARED((NP,), jnp.float32),
        pltpu.SemaphoreType.DMA((NSLOT,)),
        pltpu.SemaphoreType.DMA((NSLOT,)),
        pltpu.SemaphoreType.DMA((NSLOT,)),
    ]
    return pl.kernel(_sc_body, out_type=out_type, mesh=mesh,
                     scratch_types=scratch)


_sc_agg = _make_sc_agg()


def _dense_body(agg2_ref, cnt2_ref, h_ref, Wl_ref, bl_ref, Wr_ref,
                gamma_ref, beta_ref, relu_ref, out_ref):
    agg = agg2_ref[0, :N] + agg2_ref[1, :N]
    cnt = cnt2_ref[0, :N] + cnt2_ref[1, :N]
    mean = agg / jnp.maximum(cnt[:, None], 1.0)
    h = h_ref[...]
    out = (jnp.dot(mean, Wl_ref[...], preferred_element_type=jnp.float32)
           + bl_ref[...][None, :]
           + jnp.dot(h, Wr_ref[...], preferred_element_type=jnp.float32))
    nrm = jnp.sqrt(jnp.sum(out * out, axis=1, keepdims=True))
    out = out / jnp.maximum(nrm, 1e-12)
    mu = jnp.mean(out, axis=0, keepdims=True)
    var = jnp.mean((out - mu) * (out - mu), axis=0, keepdims=True)
    out = (gamma_ref[...][None, :] * (out - mu) / jnp.sqrt(var + 1e-5)
           + beta_ref[...][None, :] + h)
    out = jnp.where(relu_ref[0] > 0.0, jnp.maximum(out, 0.0), out)
    out_ref[...] = out


_dense = pl.pallas_call(
    _dense_body, out_shape=jax.ShapeDtypeStruct((N, D), jnp.float32))


def kernel(x, edge_index, Wl, bl, Wr, gamma, beta):
    src = jnp.concatenate(
        [edge_index[0].reshape(NW, EPW),
         jnp.zeros((NW, PADE), jnp.int32)], axis=1).reshape(-1)
    dst = jnp.concatenate(
        [edge_index[1].reshape(NW, EPW),
         jnp.full((NW, PADE), NP - 1, jnp.int32)], axis=1).reshape(-1)
    relu_flags = jnp.arange(L, dtype=jnp.float32)[::-1].reshape(L, 1)

    def _layer(h, xs):
        Wl_i, bl_i, Wr_i, gamma_i, beta_i, relu_i = xs
        agg2, cnt2 = _sc_agg(h, src, dst)
        h = _dense(agg2, cnt2.reshape(NC, NP), h, Wl_i, bl_i, Wr_i,
                   gamma_i, beta_i, relu_i)
        return h, None

    h, _ = lax.scan(_layer, x,
                    (Wl[:L], bl[:L], Wr[:L], gamma[:L], beta[:L], relu_flags))
    return h
